# baseline (speedup 1.0000x reference)
import functools

import jax
import jax.numpy as jnp
from jax import lax
from jax.experimental import pallas as pl
from jax.experimental.pallas import tpu as pltpu
from jax.experimental.pallas import tpu_sc as plsc

_C = 32


def _unnorm(coord, size, align_corners):
    if align_corners:
        return (coord + 1.0) * 0.5 * (size - 1.0)
    return ((coord + 1.0) * size - 1.0) * 0.5


def _reflect(coord, twice_low, twice_high):
    if twice_low == twice_high:
        return jnp.zeros_like(coord)
    mn = twice_low * 0.5
    span = (twice_high - twice_low) * 0.5
    c = jnp.abs(coord - mn)
    extra = jnp.mod(c, span)
    flips = jnp.floor(c / span)
    return jnp.where(jnp.mod(flips, 2.0) == 0.0, extra + mn, span - extra + mn)


def _coord(coord, size, padding_mode, align_corners):
    if padding_mode == 'border':
        return jnp.clip(coord, 0.0, size - 1.0)
    if padding_mode == 'reflection':
        if align_corners:
            coord = _reflect(coord, 0.0, 2.0 * (size - 1.0))
        else:
            coord = _reflect(coord, -1.0, 2.0 * size - 1.0)
        return jnp.clip(coord, 0.0, size - 1.0)
    return coord


def _cubic_w(t):
    A = -0.75
    def k1(v):
        return ((A + 2.0) * v - (A + 3.0)) * v * v + 1.0
    def k2(v):
        return ((A * v - 5.0 * A) * v + 8.0 * A) * v - 4.0 * A
    return [k2(t + 1.0), k1(t), k1(1.0 - t), k2(2.0 - t)]


def _taps_2d(gr, mode, pad, ac, H, W):
    ix = _unnorm(gr[..., 0], float(W), ac)
    iy = _unnorm(gr[..., 1], float(H), ac)
    if mode != 'bicubic':
        ix = _coord(ix, float(W), pad, ac)
        iy = _coord(iy, float(H), pad, ac)
    taps = []
    if mode == 'bilinear':
        x0 = jnp.floor(ix); y0 = jnp.floor(iy)
        tx = ix - x0; ty = iy - y0
        x0i = x0.astype(jnp.int32); y0i = y0.astype(jnp.int32)
        taps = [(x0i, y0i, (1.0 - tx) * (1.0 - ty)),
                (x0i + 1, y0i, tx * (1.0 - ty)),
                (x0i, y0i + 1, (1.0 - tx) * ty),
                (x0i + 1, y0i + 1, tx * ty)]
    elif mode == 'nearest':
        taps = [(jnp.round(ix).astype(jnp.int32),
                 jnp.round(iy).astype(jnp.int32), None)]
    else:
        x0 = jnp.floor(ix); y0 = jnp.floor(iy)
        tx = ix - x0; ty = iy - y0
        wx = _cubic_w(tx); wy = _cubic_w(ty)
        for j in range(4):
            cy = _coord(y0 + (j - 1.0), float(H), pad, ac)
            cyi = jnp.round(cy).astype(jnp.int32)
            for i in range(4):
                cx = _coord(x0 + (i - 1.0), float(W), pad, ac)
                cxi = jnp.round(cx).astype(jnp.int32)
                taps.append((cxi, cyi, wx[i] * wy[j]))
    N = gr.shape[0]
    b = (jnp.arange(N, dtype=jnp.int32) * (H * W)).reshape(N, 1, 1)
    idxs, ws = [], []
    for (xi, yi, w) in taps:
        m = (xi >= 0) & (xi < W) & (yi >= 0) & (yi < H)
        fi = b + jnp.clip(yi, 0, H - 1) * W + jnp.clip(xi, 0, W - 1)
        wt = m.astype(jnp.float32) if w is None else w * m
        idxs.append(fi)
        ws.append(wt)
    K = len(taps)
    idx = jnp.stack(idxs, axis=-1).reshape(-1, K).astype(jnp.int32)
    wgt = jnp.stack(ws, axis=-1).reshape(-1, K).astype(jnp.float32)
    return idx, wgt


def _taps_3d(gr, mode, pad, ac, D, H, W):
    ix = _coord(_unnorm(gr[..., 0], float(W), ac), float(W), pad, ac)
    iy = _coord(_unnorm(gr[..., 1], float(H), ac), float(H), pad, ac)
    iz = _coord(_unnorm(gr[..., 2], float(D), ac), float(D), pad, ac)
    taps = []
    if mode == 'bilinear':
        x0 = jnp.floor(ix); y0 = jnp.floor(iy); z0 = jnp.floor(iz)
        tx = ix - x0; ty = iy - y0; tz = iz - z0
        x0i = x0.astype(jnp.int32); y0i = y0.astype(jnp.int32); z0i = z0.astype(jnp.int32)
        for dz in range(2):
            wz = tz if dz == 1 else (1.0 - tz)
            for dy in range(2):
                wy = ty if dy == 1 else (1.0 - ty)
                for dx in range(2):
                    wx = tx if dx == 1 else (1.0 - tx)
                    taps.append((x0i + dx, y0i + dy, z0i + dz, wx * wy * wz))
    else:
        taps = [(jnp.round(ix).astype(jnp.int32),
                 jnp.round(iy).astype(jnp.int32),
                 jnp.round(iz).astype(jnp.int32), None)]
    N = gr.shape[0]
    b = (jnp.arange(N, dtype=jnp.int32) * (D * H * W)).reshape(N, 1, 1, 1)
    idxs, ws = [], []
    for (xi, yi, zi, w) in taps:
        m = (xi >= 0) & (xi < W) & (yi >= 0) & (yi < H) & (zi >= 0) & (zi < D)
        fi = (b + jnp.clip(zi, 0, D - 1) * (H * W)
              + jnp.clip(yi, 0, H - 1) * W + jnp.clip(xi, 0, W - 1))
        wt = m.astype(jnp.float32) if w is None else w * m
        idxs.append(fi)
        ws.append(wt)
    K = len(taps)
    idx = jnp.stack(idxs, axis=-1).reshape(-1, K).astype(jnp.int32)
    wgt = jnp.stack(ws, axis=-1).reshape(-1, K).astype(jnp.float32)
    return idx, wgt


@functools.lru_cache(maxsize=None)
def _make_stage_simple(P, K, Bc):
    info = plsc.get_sparse_core_info()
    NC, NS, NL = info.num_cores, info.num_subcores, info.num_lanes
    NW = NC * NS
    R = Bc * K
    RC = R // 128
    Pw = P // NW
    nchunks = Pw // Bc
    mesh = plsc.VectorSubcoreMesh(core_axis_name="c", subcore_axis_name="s")

    @functools.partial(
        pl.kernel,
        mesh=mesh,
        compiler_params=pltpu.CompilerParams(needs_layout_passes=False,
                                             use_tc_tiling_on_sc=False),
        out_type=jax.ShapeDtypeStruct((P, _C), jnp.float32),
        scratch_types=[
            pltpu.VMEM((RC, 128), jnp.int32),
            pltpu.VMEM((R, _C), jnp.float32),
            pltpu.VMEM((K, Bc), jnp.float32),
            pltpu.VMEM((Bc, _C), jnp.float32),
            pltpu.SemaphoreType.DMA,
        ],
    )
    def stage(tbl, idx_hbm, w_hbm, out_hbm, idx_v, rows_v, w_v, o_v, sem):
        wid = lax.axis_index("s") * NC + lax.axis_index("c")

        def chunk(i, carry):
            cid = wid * nchunks + i
            pbase = cid * Bc
            pltpu.sync_copy(idx_hbm.at[pl.ds(cid * RC, RC)], idx_v)
            pltpu.sync_copy(w_hbm.at[cid], w_v)
            cps = [pltpu.async_copy(tbl.at[idx_v.at[j]],
                                    rows_v.at[pl.ds(j * 128, 128)], sem)
                   for j in range(RC)]
            for cp in cps:
                cp.wait()

            def group(g, carry2):
                pos16 = jnp.full((NL,), g * NL, jnp.int32) + lax.iota(jnp.int32, NL)
                base = pos16 * K

                def chan(c, carry3):
                    cv = jnp.full((NL,), c, jnp.int32)
                    acc = jnp.zeros((NL,), jnp.float32)
                    for k in range(K):
                        vals = plsc.load_gather(rows_v, [base + k, cv])
                        wk = w_v[k, pl.ds(g * NL, NL)]
                        acc = acc + wk * vals
                    plsc.store_scatter(o_v, [pos16, cv], acc)
                    return carry3

                return lax.fori_loop(0, _C, chan, carry2)

            lax.fori_loop(0, Bc // NL, group, 0)
            pltpu.sync_copy(o_v, out_hbm.at[pl.ds(pbase, Bc)])
            return carry

        lax.fori_loop(0, nchunks, chunk, 0)

    return stage


@functools.lru_cache(maxsize=None)
def _make_stage(P, K, Bc):
    info = plsc.get_sparse_core_info()
    NC, NS, NL = info.num_cores, info.num_subcores, info.num_lanes
    NW = NC * NS
    R = Bc * K
    RC = R // 128
    Pw = P // NW
    nchunks = Pw // Bc
    assert Pw * NW == P and nchunks * Bc == Pw and RC * 128 == R
    assert nchunks % 2 == 0

    mesh = plsc.VectorSubcoreMesh(core_axis_name="c", subcore_axis_name="s")

    @functools.partial(
        pl.kernel,
        mesh=mesh,
        compiler_params=pltpu.CompilerParams(needs_layout_passes=False,
                                             use_tc_tiling_on_sc=False),
        out_type=jax.ShapeDtypeStruct((P, _C), jnp.float32),
        scratch_types=[
            pltpu.VMEM((2 * RC, 128), jnp.int32),
            pltpu.VMEM((2 * R, _C), jnp.float32),
            pltpu.VMEM((2 * K, Bc), jnp.float32),
            pltpu.VMEM((2 * Bc, _C), jnp.float32),
            pltpu.SemaphoreType.DMA, pltpu.SemaphoreType.DMA,
            pltpu.SemaphoreType.DMA, pltpu.SemaphoreType.DMA,
            pltpu.SemaphoreType.DMA, pltpu.SemaphoreType.DMA,
        ],
    )
    def stage(tbl, idx_hbm, w_hbm, out_hbm, idx_v, rows_v, w_v, o_v,
              sp0, sp1, sg0, sg1, so0, so1):
        wid = lax.axis_index("s") * NC + lax.axis_index("c")
        c0 = wid * nchunks
        sp = (sp0, sp1)
        sg = (sg0, sg1)
        so = (so0, so1)

        def start_pre(cid, s):
            pltpu.async_copy(idx_hbm.at[pl.ds(cid * RC, RC)],
                             idx_v.at[pl.ds(s * RC, RC)], sp[s])
            pltpu.async_copy(w_hbm.at[cid], w_v.at[pl.ds(s * K, K)], sp[s])

        def wait_pre(s):
            pltpu.make_async_copy(idx_hbm.at[pl.ds(0, RC)],
                                  idx_v.at[pl.ds(s * RC, RC)], sp[s]).wait()
            pltpu.make_async_copy(w_hbm.at[0], w_v.at[pl.ds(s * K, K)], sp[s]).wait()

        def fire_g(s):
            for j in range(RC):
                pltpu.async_copy(tbl.at[idx_v.at[s * RC + j]],
                                 rows_v.at[pl.ds(s * R + j * 128, 128)], sg[s])

        def wait_g(s):
            for j in range(RC):
                pltpu.make_async_copy(tbl.at[idx_v.at[s * RC + j]],
                                      rows_v.at[pl.ds(s * R + j * 128, 128)],
                                      sg[s]).wait()

        def start_out(cid, s):
            pltpu.async_copy(o_v.at[pl.ds(s * Bc, Bc)],
                             out_hbm.at[pl.ds(cid * Bc, Bc)], so[s])

        def wait_out(s):
            pltpu.make_async_copy(o_v.at[pl.ds(s * Bc, Bc)],
                                  out_hbm.at[pl.ds(0, Bc)], so[s]).wait()

        def compute(s):
            def group(g, carry2):
                pos16 = jnp.full((NL,), g * NL, jnp.int32) + lax.iota(jnp.int32, NL)
                base = pos16 * K + (s * R)
                opos = pos16 + (s * Bc)

                def chan(c, carry3):
                    cv = jnp.full((NL,), c, jnp.int32)
                    acc = jnp.zeros((NL,), jnp.float32)
                    for k in range(K):
                        vals = plsc.load_gather(rows_v, [base + k, cv])
                        wk = w_v[s * K + k, pl.ds(g * NL, NL)]
                        acc = acc + wk * vals
                    plsc.store_scatter(o_v, [opos, cv], acc)
                    return carry3

                return lax.fori_loop(0, _C, chan, carry2)

            lax.fori_loop(0, Bc // NL, group, 0)

        start_pre(c0, 0)
        start_pre(c0 + 1, 1)
        wait_pre(0)
        fire_g(0)

        def pair(i2, carry):
            c = 2 * i2
            wait_pre(1)
            fire_g(1)
            wait_g(0)

            @pl.when(c + 2 < nchunks)
            def _():
                start_pre(c0 + c + 2, 0)

            @pl.when(c >= 2)
            def _():
                wait_out(0)

            compute(0)
            start_out(c0 + c, 0)
            @pl.when(c + 2 < nchunks)
            def _():
                wait_pre(0)
                fire_g(0)

            wait_g(1)

            @pl.when(c + 3 < nchunks)
            def _():
                start_pre(c0 + c + 3, 1)

            @pl.when(c >= 2)
            def _():
                wait_out(1)

            compute(1)
            start_out(c0 + c + 1, 1)
            return carry

        lax.fori_loop(0, nchunks // 2, pair, 0)
        wait_out(0)
        wait_out(1)

    return stage


def _fold_nearest(stages):
    out = []
    perm = None
    for idx, w in stages:
        if idx.shape[1] == 1:
            n, s = idx[:, 0], w[:, 0]
            if perm is None:
                perm = (n, s)
            else:
                m0, sig0 = perm
                perm = (jnp.take(m0, n), s * jnp.take(sig0, n))
        else:
            if perm is not None:
                m, sig = perm
                w = w * jnp.take(sig, idx)
                idx = jnp.take(m, idx)
                perm = None
            out.append((idx, w))
    if perm is not None:
        m, sig = perm
        idx, w = out[-1]
        out[-1] = (jnp.take(idx, m, axis=0),
                   sig[:, None] * jnp.take(w, m, axis=0))
    return out


def _run_stage(t, idx, w):
    P, K = idx.shape
    Bc = 128 if K == 8 else 64
    idx_r = idx.reshape(P * K // 128, 128)
    w_r = w.reshape(P // Bc, Bc, K).swapaxes(1, 2)
    return _make_stage_simple(P, K, Bc)(t, idx_r, w_r)


_SPECS_2D = [('bilinear', 'zeros', False), ('bilinear', 'border', False), ('bilinear', 'reflection', False),
             ('nearest', 'zeros', False), ('nearest', 'border', False), ('nearest', 'reflection', False),
             ('bicubic', 'zeros', False), ('bicubic', 'border', False), ('bicubic', 'reflection', False),
             ('bilinear', 'zeros', True), ('bilinear', 'border', True), ('bilinear', 'reflection', True),
             ('nearest', 'zeros', True), ('nearest', 'border', True), ('nearest', 'reflection', True),
             ('bicubic', 'zeros', True), ('bicubic', 'border', True), ('bicubic', 'reflection', True)]
_SPECS_3D = [('bilinear', 'zeros', False), ('bilinear', 'border', False), ('bilinear', 'reflection', False),
             ('nearest', 'zeros', False), ('nearest', 'border', False), ('nearest', 'reflection', False),
             ('bilinear', 'zeros', True), ('bilinear', 'border', True), ('bilinear', 'reflection', True),
             ('nearest', 'zeros', True), ('nearest', 'border', True), ('nearest', 'reflection', True)]


def kernel(x, xg1, xg2, y, yg1, yg2):
    N2, C2, H2, W2 = x.shape
    t2 = x.transpose(0, 2, 3, 1).reshape(N2 * H2 * W2, C2)
    g2 = [xg1, xg2]
    st2 = [_taps_2d(g2[i % 2], m, p, ac, H2, W2)
           for i, (m, p, ac) in enumerate(_SPECS_2D)]
    for idx, w in _fold_nearest(st2):
        t2 = _run_stage(t2, idx, w)
    x_out = t2.reshape(N2, H2, W2, C2).transpose(0, 3, 1, 2)

    N3, C3, D3, H3, W3 = y.shape
    t3 = y.transpose(0, 2, 3, 4, 1).reshape(N3 * D3 * H3 * W3, C3)
    g3 = [yg1, yg2]
    st3 = [_taps_3d(g3[i % 2], m, p, ac, D3, H3, W3)
           for i, (m, p, ac) in enumerate(_SPECS_3D)]
    for idx, w in _fold_nearest(st3):
        t3 = _run_stage(t3, idx, w)
    y_out = t3.reshape(N3, D3, H3, W3, C3).transpose(0, 4, 1, 2, 3)

    return (x_out, y_out)


# speedup vs baseline: 5.8003x; 5.8003x over previous
import functools

import jax
import jax.numpy as jnp
from jax import lax
from jax.experimental import pallas as pl
from jax.experimental.pallas import tpu as pltpu
from jax.experimental.pallas import tpu_sc as plsc

_C = 32


def _unnorm(coord, size, align_corners):
    if align_corners:
        return (coord + 1.0) * 0.5 * (size - 1.0)
    return ((coord + 1.0) * size - 1.0) * 0.5


def _reflect(coord, twice_low, twice_high):
    if twice_low == twice_high:
        return jnp.zeros_like(coord)
    mn = twice_low * 0.5
    span = (twice_high - twice_low) * 0.5
    c = jnp.abs(coord - mn)
    extra = jnp.mod(c, span)
    flips = jnp.floor(c / span)
    return jnp.where(jnp.mod(flips, 2.0) == 0.0, extra + mn, span - extra + mn)


def _coord(coord, size, padding_mode, align_corners):
    if padding_mode == 'border':
        return jnp.clip(coord, 0.0, size - 1.0)
    if padding_mode == 'reflection':
        if align_corners:
            coord = _reflect(coord, 0.0, 2.0 * (size - 1.0))
        else:
            coord = _reflect(coord, -1.0, 2.0 * size - 1.0)
        return jnp.clip(coord, 0.0, size - 1.0)
    return coord


def _cubic_w(t):
    A = -0.75
    def k1(v):
        return ((A + 2.0) * v - (A + 3.0)) * v * v + 1.0
    def k2(v):
        return ((A * v - 5.0 * A) * v + 8.0 * A) * v - 4.0 * A
    return [k2(t + 1.0), k1(t), k1(1.0 - t), k2(2.0 - t)]


def _taps_2d(gr, mode, pad, ac, H, W):
    ix = _unnorm(gr[..., 0], float(W), ac)
    iy = _unnorm(gr[..., 1], float(H), ac)
    if mode != 'bicubic':
        ix = _coord(ix, float(W), pad, ac)
        iy = _coord(iy, float(H), pad, ac)
    taps = []
    if mode == 'bilinear':
        x0 = jnp.floor(ix); y0 = jnp.floor(iy)
        tx = ix - x0; ty = iy - y0
        x0i = x0.astype(jnp.int32); y0i = y0.astype(jnp.int32)
        taps = [(x0i, y0i, (1.0 - tx) * (1.0 - ty)),
                (x0i + 1, y0i, tx * (1.0 - ty)),
                (x0i, y0i + 1, (1.0 - tx) * ty),
                (x0i + 1, y0i + 1, tx * ty)]
    elif mode == 'nearest':
        taps = [(jnp.round(ix).astype(jnp.int32),
                 jnp.round(iy).astype(jnp.int32), None)]
    else:
        x0 = jnp.floor(ix); y0 = jnp.floor(iy)
        tx = ix - x0; ty = iy - y0
        wx = _cubic_w(tx); wy = _cubic_w(ty)
        for j in range(4):
            cy = _coord(y0 + (j - 1.0), float(H), pad, ac)
            cyi = jnp.round(cy).astype(jnp.int32)
            for i in range(4):
                cx = _coord(x0 + (i - 1.0), float(W), pad, ac)
                cxi = jnp.round(cx).astype(jnp.int32)
                taps.append((cxi, cyi, wx[i] * wy[j]))
    N = gr.shape[0]
    b = (jnp.arange(N, dtype=jnp.int32) * (H * W)).reshape(N, 1, 1)
    idxs, ws = [], []
    for (xi, yi, w) in taps:
        m = (xi >= 0) & (xi < W) & (yi >= 0) & (yi < H)
        fi = b + jnp.clip(yi, 0, H - 1) * W + jnp.clip(xi, 0, W - 1)
        wt = m.astype(jnp.float32) if w is None else w * m
        idxs.append(fi)
        ws.append(wt)
    K = len(taps)
    idx = jnp.stack(idxs, axis=-1).reshape(-1, K).astype(jnp.int32)
    wgt = jnp.stack(ws, axis=-1).reshape(-1, K).astype(jnp.float32)
    return idx, wgt


def _taps_3d(gr, mode, pad, ac, D, H, W):
    ix = _coord(_unnorm(gr[..., 0], float(W), ac), float(W), pad, ac)
    iy = _coord(_unnorm(gr[..., 1], float(H), ac), float(H), pad, ac)
    iz = _coord(_unnorm(gr[..., 2], float(D), ac), float(D), pad, ac)
    taps = []
    if mode == 'bilinear':
        x0 = jnp.floor(ix); y0 = jnp.floor(iy); z0 = jnp.floor(iz)
        tx = ix - x0; ty = iy - y0; tz = iz - z0
        x0i = x0.astype(jnp.int32); y0i = y0.astype(jnp.int32); z0i = z0.astype(jnp.int32)
        for dz in range(2):
            wz = tz if dz == 1 else (1.0 - tz)
            for dy in range(2):
                wy = ty if dy == 1 else (1.0 - ty)
                for dx in range(2):
                    wx = tx if dx == 1 else (1.0 - tx)
                    taps.append((x0i + dx, y0i + dy, z0i + dz, wx * wy * wz))
    else:
        taps = [(jnp.round(ix).astype(jnp.int32),
                 jnp.round(iy).astype(jnp.int32),
                 jnp.round(iz).astype(jnp.int32), None)]
    N = gr.shape[0]
    b = (jnp.arange(N, dtype=jnp.int32) * (D * H * W)).reshape(N, 1, 1, 1)
    idxs, ws = [], []
    for (xi, yi, zi, w) in taps:
        m = (xi >= 0) & (xi < W) & (yi >= 0) & (yi < H) & (zi >= 0) & (zi < D)
        fi = (b + jnp.clip(zi, 0, D - 1) * (H * W)
              + jnp.clip(yi, 0, H - 1) * W + jnp.clip(xi, 0, W - 1))
        wt = m.astype(jnp.float32) if w is None else w * m
        idxs.append(fi)
        ws.append(wt)
    K = len(taps)
    idx = jnp.stack(idxs, axis=-1).reshape(-1, K).astype(jnp.int32)
    wgt = jnp.stack(ws, axis=-1).reshape(-1, K).astype(jnp.float32)
    return idx, wgt


@functools.lru_cache(maxsize=None)
def _make_stage_simple(P, K, Bc):
    info = plsc.get_sparse_core_info()
    NC, NS, NL = info.num_cores, info.num_subcores, info.num_lanes
    NW = NC * NS
    R = Bc * K
    RC = R // 128
    Pw = P // NW
    nchunks = Pw // Bc
    mesh = plsc.VectorSubcoreMesh(core_axis_name="c", subcore_axis_name="s")

    @functools.partial(
        pl.kernel,
        mesh=mesh,
        compiler_params=pltpu.CompilerParams(needs_layout_passes=False,
                                             use_tc_tiling_on_sc=False),
        out_type=jax.ShapeDtypeStruct((P, _C), jnp.float32),
        scratch_types=[
            pltpu.VMEM((RC, 128), jnp.int32),
            pltpu.VMEM((R, _C), jnp.float32),
            pltpu.VMEM((K, Bc), jnp.float32),
            pltpu.VMEM((Bc, _C), jnp.float32),
            pltpu.SemaphoreType.DMA,
        ],
    )
    def stage(tbl, idx_hbm, w_hbm, out_hbm, idx_v, rows_v, w_v, o_v, sem):
        wid = lax.axis_index("s") * NC + lax.axis_index("c")

        def chunk(i, carry):
            cid = wid * nchunks + i
            pbase = cid * Bc
            pltpu.sync_copy(idx_hbm.at[pl.ds(cid * RC, RC)], idx_v)
            pltpu.sync_copy(w_hbm.at[cid], w_v)
            cps = [pltpu.async_copy(tbl.at[idx_v.at[j]],
                                    rows_v.at[pl.ds(j * 128, 128)], sem)
                   for j in range(RC)]
            for cp in cps:
                cp.wait()

            def group(g, carry2):
                pos16 = jnp.full((NL,), g * NL, jnp.int32) + lax.iota(jnp.int32, NL)
                base = pos16 * K

                def chan(c, carry3):
                    cv = jnp.full((NL,), c, jnp.int32)
                    acc = jnp.zeros((NL,), jnp.float32)
                    for k in range(K):
                        vals = plsc.load_gather(rows_v, [base + k, cv])
                        wk = w_v[k, pl.ds(g * NL, NL)]
                        acc = acc + wk * vals
                    plsc.store_scatter(o_v, [pos16, cv], acc)
                    return carry3

                return lax.fori_loop(0, _C, chan, carry2)

            lax.fori_loop(0, Bc // NL, group, 0)
            pltpu.sync_copy(o_v, out_hbm.at[pl.ds(pbase, Bc)])
            return carry

        lax.fori_loop(0, nchunks, chunk, 0)

    return stage


@functools.lru_cache(maxsize=None)
def _make_stage(P, K, Bc):
    info = plsc.get_sparse_core_info()
    NC, NS, NL = info.num_cores, info.num_subcores, info.num_lanes
    NW = NC * NS
    R = Bc * K
    RC = R // 128
    Pw = P // NW
    nchunks = Pw // Bc
    assert Pw * NW == P and nchunks * Bc == Pw and RC * 128 == R
    assert nchunks % 2 == 0

    mesh = plsc.VectorSubcoreMesh(core_axis_name="c", subcore_axis_name="s")

    @functools.partial(
        pl.kernel,
        mesh=mesh,
        compiler_params=pltpu.CompilerParams(needs_layout_passes=False,
                                             use_tc_tiling_on_sc=False),
        out_type=jax.ShapeDtypeStruct((P, _C), jnp.float32),
        scratch_types=[
            pltpu.VMEM((2 * RC, 128), jnp.int32),
            pltpu.VMEM((2 * R, _C), jnp.float32),
            pltpu.VMEM((2 * K, Bc), jnp.float32),
            pltpu.VMEM((2 * Bc, _C), jnp.float32),
            pltpu.SemaphoreType.DMA, pltpu.SemaphoreType.DMA,
            pltpu.SemaphoreType.DMA, pltpu.SemaphoreType.DMA,
            pltpu.SemaphoreType.DMA, pltpu.SemaphoreType.DMA,
            pltpu.SemaphoreType.DMA, pltpu.SemaphoreType.DMA,
        ],
    )
    def stage(tbl, idx_hbm, w_hbm, out_hbm, idx_v, rows_v, w_v, o_v,
              si0, si1, sw0, sw1, sg0, sg1, so0, so1):
        wid = lax.axis_index("s") * NC + lax.axis_index("c")
        c0 = wid * nchunks
        si = (si0, si1)
        sw = (sw0, sw1)
        sg = (sg0, sg1)
        so = (so0, so1)

        def start_idx(cid, s):
            pltpu.async_copy(idx_hbm.at[pl.ds(cid * RC, RC)],
                             idx_v.at[pl.ds(s * RC, RC)], si[s])

        def wait_idx(s):
            pltpu.make_async_copy(idx_hbm.at[pl.ds(0, RC)],
                                  idx_v.at[pl.ds(s * RC, RC)], si[s]).wait()

        def start_w(cid, s):
            pltpu.async_copy(w_hbm.at[cid], w_v.at[pl.ds(s * K, K)], sw[s])

        def wait_w(s):
            pltpu.make_async_copy(w_hbm.at[0], w_v.at[pl.ds(s * K, K)], sw[s]).wait()

        def fire_g(s):
            for j in range(RC):
                pltpu.async_copy(tbl.at[idx_v.at[s * RC + j]],
                                 rows_v.at[pl.ds(s * R + j * 128, 128)], sg[s])

        def wait_g(s):
            for j in range(RC):
                pltpu.make_async_copy(tbl.at[idx_v.at[s * RC + j]],
                                      rows_v.at[pl.ds(s * R + j * 128, 128)],
                                      sg[s]).wait()

        def start_out(cid, s):
            pltpu.async_copy(o_v.at[pl.ds(s * Bc, Bc)],
                             out_hbm.at[pl.ds(cid * Bc, Bc)], so[s])

        def wait_out(s):
            pltpu.make_async_copy(o_v.at[pl.ds(s * Bc, Bc)],
                                  out_hbm.at[pl.ds(0, Bc)], so[s]).wait()

        def compute(s):
            def group(g, carry2):
                pos16 = jnp.full((NL,), g * NL, jnp.int32) + lax.iota(jnp.int32, NL)
                base = pos16 * K + (s * R)
                opos = pos16 + (s * Bc)

                def chan(c, carry3):
                    cv = jnp.full((NL,), c, jnp.int32)
                    acc = jnp.zeros((NL,), jnp.float32)
                    for k in range(K):
                        vals = plsc.load_gather(rows_v, [base + k, cv])
                        wk = w_v[s * K + k, pl.ds(g * NL, NL)]
                        acc = acc + wk * vals
                    plsc.store_scatter(o_v, [opos, cv], acc)
                    return carry3

                return lax.fori_loop(0, _C, chan, carry2)

            lax.fori_loop(0, Bc // NL, group, 0)

        start_idx(c0, 0)
        start_idx(c0 + 1, 1)
        start_w(c0, 0)
        start_w(c0 + 1, 1)
        wait_idx(0)
        fire_g(0)

        def pair(i2, carry):
            c = 2 * i2
            wait_idx(1)
            fire_g(1)
            wait_g(0)

            @pl.when(c + 2 < nchunks)
            def _():
                start_idx(c0 + c + 2, 0)

            @pl.when(c >= 2)
            def _():
                wait_out(0)

            wait_w(0)
            compute(0)
            start_out(c0 + c, 0)

            @pl.when(c + 2 < nchunks)
            def _():
                start_w(c0 + c + 2, 0)

            @pl.when(c + 2 < nchunks)
            def _():
                wait_idx(0)
                fire_g(0)

            wait_g(1)

            @pl.when(c + 3 < nchunks)
            def _():
                start_idx(c0 + c + 3, 1)

            @pl.when(c >= 2)
            def _():
                wait_out(1)

            wait_w(1)
            compute(1)
            start_out(c0 + c + 1, 1)

            @pl.when(c + 3 < nchunks)
            def _():
                start_w(c0 + c + 3, 1)

            return carry

        lax.fori_loop(0, nchunks // 2, pair, 0)
        wait_out(0)
        wait_out(1)

    return stage


def _fold_nearest(stages):
    out = []
    perm = None
    for idx, w in stages:
        if idx.shape[1] == 1:
            n, s = idx[:, 0], w[:, 0]
            if perm is None:
                perm = (n, s)
            else:
                m0, sig0 = perm
                perm = (jnp.take(m0, n), s * jnp.take(sig0, n))
        else:
            if perm is not None:
                m, sig = perm
                w = w * jnp.take(sig, idx)
                idx = jnp.take(m, idx)
                perm = None
            out.append((idx, w))
    if perm is not None:
        m, sig = perm
        idx, w = out[-1]
        out[-1] = (jnp.take(idx, m, axis=0),
                   sig[:, None] * jnp.take(w, m, axis=0))
    return out


def _run_stage(t, idx, w):
    P, K = idx.shape
    if K == 1:
        Bc = 896 if P % 896 == 0 else 1024
        make = _make_stage_simple
    else:
        Bc = 128 if K == 8 else 64
        make = _make_stage
    idx_r = idx.reshape(P * K // 128, 128)
    w_r = w.reshape(P // Bc, Bc, K).swapaxes(1, 2)
    return make(P, K, Bc)(t, idx_r, w_r)


_SPECS_2D = [('bilinear', 'zeros', False), ('bilinear', 'border', False), ('bilinear', 'reflection', False),
             ('nearest', 'zeros', False), ('nearest', 'border', False), ('nearest', 'reflection', False),
             ('bicubic', 'zeros', False), ('bicubic', 'border', False), ('bicubic', 'reflection', False),
             ('bilinear', 'zeros', True), ('bilinear', 'border', True), ('bilinear', 'reflection', True),
             ('nearest', 'zeros', True), ('nearest', 'border', True), ('nearest', 'reflection', True),
             ('bicubic', 'zeros', True), ('bicubic', 'border', True), ('bicubic', 'reflection', True)]
_SPECS_3D = [('bilinear', 'zeros', False), ('bilinear', 'border', False), ('bilinear', 'reflection', False),
             ('nearest', 'zeros', False), ('nearest', 'border', False), ('nearest', 'reflection', False),
             ('bilinear', 'zeros', True), ('bilinear', 'border', True), ('bilinear', 'reflection', True),
             ('nearest', 'zeros', True), ('nearest', 'border', True), ('nearest', 'reflection', True)]


def kernel(x, xg1, xg2, y, yg1, yg2):
    N2, C2, H2, W2 = x.shape
    t2 = x.transpose(0, 2, 3, 1).reshape(N2 * H2 * W2, C2)
    g2 = [xg1, xg2]
    for i, (m, p, ac) in enumerate(_SPECS_2D):
        idx, w = _taps_2d(g2[i % 2], m, p, ac, H2, W2)
        t2 = _run_stage(t2, idx, w)
    x_out = t2.reshape(N2, H2, W2, C2).transpose(0, 3, 1, 2)

    N3, C3, D3, H3, W3 = y.shape
    t3 = y.transpose(0, 2, 3, 4, 1).reshape(N3 * D3 * H3 * W3, C3)
    g3 = [yg1, yg2]
    for i, (m, p, ac) in enumerate(_SPECS_3D):
        idx, w = _taps_3d(g3[i % 2], m, p, ac, D3, H3, W3)
        t3 = _run_stage(t3, idx, w)
    y_out = t3.reshape(N3, D3, H3, W3, C3).transpose(0, 4, 1, 2, 3)

    return (x_out, y_out)
